# baseline (speedup 1.0000x reference)
import functools

import jax
import jax.numpy as jnp
from jax.experimental import pallas as pl

B, T, D = 1, 2048, 768
DFF = 4 * D
N_SHARED = 2
N_PRIVATE = 8
TOP_K = 2

BT = 512
BF = 1536
N_TB = T // BT
N_FB = DFF // BF
N_E = N_SHARED + N_PRIVATE


def _top2_weights(logits, i):
    n = logits.shape[-1]
    iota = jax.lax.broadcasted_iota(jnp.int32, logits.shape, 1)
    m1 = jnp.max(logits, axis=-1, keepdims=True)
    a1 = jnp.min(jnp.where(logits == m1, iota, n), axis=-1, keepdims=True)
    masked = jnp.where(iota == a1, -jnp.inf, logits)
    m2 = jnp.max(masked, axis=-1, keepdims=True)
    a2 = jnp.min(jnp.where(masked == m2, iota, n), axis=-1, keepdims=True)
    s1 = 1.0 / (1.0 + jnp.exp(m2 - m1))
    s2 = 1.0 - s1
    return jnp.where(a1 == i, s1, 0.0) + jnp.where(a2 == i, s2, 0.0)


def _moe_kernel(x_ref, ws1_ref, ws2_ref, wp1_ref, wp2_ref, wr_ref, out_ref):
    e = pl.program_id(1)
    dblk = pl.program_id(2)

    @pl.when((e == 0) & (dblk == 0))
    def _():
        out_ref[...] = jnp.zeros_like(out_ref)

    xb = x_ref[...]

    @pl.when(e < N_SHARED)
    def _():
        h = jax.nn.gelu(
            jnp.dot(xb, ws1_ref[0], preferred_element_type=jnp.float32))
        contrib = jnp.dot(h, ws2_ref[0], preferred_element_type=jnp.float32)
        out_ref[...] += contrib * (1.0 / N_SHARED)

    @pl.when(e >= N_SHARED)
    def _():
        logits = jnp.dot(xb, wr_ref[...], preferred_element_type=jnp.float32)
        w = _top2_weights(logits, e - N_SHARED)
        h = jax.nn.gelu(
            jnp.dot(xb, wp1_ref[0], preferred_element_type=jnp.float32))
        contrib = jnp.dot(h, wp2_ref[0], preferred_element_type=jnp.float32)
        out_ref[...] += contrib * w


@functools.partial(jax.jit, static_argnames=("interpret",))
def kernel(x, Ws1, Ws2, Wp1, Wp2, Wr, interpret=False):
    x2 = x.reshape(T, D)
    out = pl.pallas_call(
        _moe_kernel,
        grid=(N_TB, N_E, N_FB),
        in_specs=[
            pl.BlockSpec((BT, D), lambda t, e, d: (t, 0)),
            pl.BlockSpec((1, D, BF), lambda t, e, d: (jnp.minimum(e, N_SHARED - 1), 0, d)),
            pl.BlockSpec((1, BF, D), lambda t, e, d: (jnp.minimum(e, N_SHARED - 1), d, 0)),
            pl.BlockSpec((1, D, BF), lambda t, e, d: (jnp.clip(e - N_SHARED, 0, N_PRIVATE - 1), 0, d)),
            pl.BlockSpec((1, BF, D), lambda t, e, d: (jnp.clip(e - N_SHARED, 0, N_PRIVATE - 1), d, 0)),
            pl.BlockSpec((D, N_PRIVATE), lambda t, e, d: (0, 0)),
        ],
        out_specs=pl.BlockSpec((BT, D), lambda t, e, d: (t, 0)),
        out_shape=jax.ShapeDtypeStruct((T, D), jnp.float32),
        interpret=interpret,
    )(x2, Ws1, Ws2, Wp1, Wp2, Wr)
    return out.reshape(B, T, D)


# speedup vs baseline: 1.2031x; 1.2031x over previous
import functools

import jax
import jax.numpy as jnp
from jax.experimental import pallas as pl

B, T, D = 1, 2048, 768
DFF = 4 * D
N_SHARED = 2
N_PRIVATE = 8
TOP_K = 2

BT = 512
BF = 1536
N_TB = T // BT
N_FB = DFF // BF
N_E = N_SHARED + N_PRIVATE


def _top2_weights(logits, i):
    n = logits.shape[-1]
    iota = jax.lax.broadcasted_iota(jnp.int32, logits.shape, 1)
    m1 = jnp.max(logits, axis=-1, keepdims=True)
    a1 = jnp.min(jnp.where(logits == m1, iota, n), axis=-1, keepdims=True)
    masked = jnp.where(iota == a1, -jnp.inf, logits)
    m2 = jnp.max(masked, axis=-1, keepdims=True)
    a2 = jnp.min(jnp.where(masked == m2, iota, n), axis=-1, keepdims=True)
    s1 = 1.0 / (1.0 + jnp.exp(m2 - m1))
    s2 = 1.0 - s1
    return jnp.where(a1 == i, s1, 0.0) + jnp.where(a2 == i, s2, 0.0)


def _moe_kernel(x_ref, ws1_ref, ws2_ref, wp1_ref, wp2_ref, wr_ref, out_ref):
    e = pl.program_id(1)
    dblk = pl.program_id(2)

    @pl.when((e == 0) & (dblk == 0))
    def _():
        out_ref[...] = jnp.zeros_like(out_ref)

    xb = x_ref[...]
    xb16 = xb.astype(jnp.bfloat16)

    @pl.when(e < N_SHARED)
    def _():
        h = jax.nn.gelu(
            jnp.dot(xb16, ws1_ref[0], preferred_element_type=jnp.float32))
        contrib = jnp.dot(h.astype(jnp.bfloat16), ws2_ref[0],
                          preferred_element_type=jnp.float32)
        out_ref[...] += contrib * (1.0 / N_SHARED)

    @pl.when(e >= N_SHARED)
    def _():
        logits = jnp.dot(xb, wr_ref[...], preferred_element_type=jnp.float32)
        w = _top2_weights(logits, e - N_SHARED)
        h = jax.nn.gelu(
            jnp.dot(xb16, wp1_ref[0], preferred_element_type=jnp.float32))
        contrib = jnp.dot(h.astype(jnp.bfloat16), wp2_ref[0],
                          preferred_element_type=jnp.float32)
        out_ref[...] += contrib * w


@functools.partial(jax.jit, static_argnames=("interpret",))
def kernel(x, Ws1, Ws2, Wp1, Wp2, Wr, interpret=False):
    x2 = x.reshape(T, D)
    Ws1 = Ws1.astype(jnp.bfloat16)
    Ws2 = Ws2.astype(jnp.bfloat16)
    Wp1 = Wp1.astype(jnp.bfloat16)
    Wp2 = Wp2.astype(jnp.bfloat16)
    out = pl.pallas_call(
        _moe_kernel,
        grid=(N_TB, N_E, N_FB),
        in_specs=[
            pl.BlockSpec((BT, D), lambda t, e, d: (t, 0)),
            pl.BlockSpec((1, D, BF), lambda t, e, d: (jnp.minimum(e, N_SHARED - 1), 0, d)),
            pl.BlockSpec((1, BF, D), lambda t, e, d: (jnp.minimum(e, N_SHARED - 1), d, 0)),
            pl.BlockSpec((1, D, BF), lambda t, e, d: (jnp.clip(e - N_SHARED, 0, N_PRIVATE - 1), 0, d)),
            pl.BlockSpec((1, BF, D), lambda t, e, d: (jnp.clip(e - N_SHARED, 0, N_PRIVATE - 1), d, 0)),
            pl.BlockSpec((D, N_PRIVATE), lambda t, e, d: (0, 0)),
        ],
        out_specs=pl.BlockSpec((BT, D), lambda t, e, d: (t, 0)),
        out_shape=jax.ShapeDtypeStruct((T, D), jnp.float32),
        interpret=interpret,
    )(x2, Ws1, Ws2, Wp1, Wp2, Wr)
    return out.reshape(B, T, D)
